# baseline (speedup 1.0000x reference)
import functools

import numpy as np
import jax
import jax.numpy as jnp
from jax import lax
from jax.experimental import pallas as pl
from jax.experimental.pallas import tpu as pltpu

_EDGES_I = [0, 1, 2, 3, 0, 5, 6, 7, 0, 9, 10, 11, 0, 13, 14, 15, 0, 17, 18, 19,
            21, 22, 23, 24, 21, 26, 27, 28, 21, 30, 31, 32, 21, 34, 35, 36, 21, 38, 39, 40]
_EDGES_J = [1, 2, 3, 4, 5, 6, 7, 8, 9, 10, 11, 12, 13, 14, 15, 16, 17, 18, 19, 20,
            22, 23, 24, 25, 26, 27, 28, 29, 30, 31, 32, 33, 34, 35, 36, 37, 38, 39, 40, 41]

_NE = len(_EDGES_I)
_NN = 42
_NC3 = _NN * 3
_B = 4096
_D = 512
_CHUNK = 128
_NBLK = _B // _CHUNK
_ITERS = 4
_MAX_CORR = 0.15
_LAM_SCALE = -1.0 / (2.0 + 1e-9)


def _edge_constants():
    gdt = np.zeros((_NC3, 3 * _NE), np.float32)
    s3t = np.zeros((3 * _NE, _NE), np.float32)
    e3 = np.zeros((_NE, 3 * _NE), np.float32)
    st = np.zeros((3 * _NE, _NC3), np.float32)
    for e in range(_NE):
        i, j = _EDGES_I[e], _EDGES_J[e]
        for c in range(3):
            gdt[3 * i + c, 3 * e + c] = 1.0
            gdt[3 * j + c, 3 * e + c] = -1.0
            s3t[3 * e + c, e] = 1.0
            e3[e, 3 * e + c] = 1.0
            st[3 * e + c, 3 * i + c] = -1.0
            st[3 * e + c, 3 * j + c] = 1.0
    return gdt, s3t, e3, st


_GDT, _S3T, _E3, _ST = _edge_constants()


def _fused_body(ht_ref, kp_ref, tau_ref, wbig_ref, bcat_ref, rl_ref,
                gdt_ref, s3t_ref, e3_ref, st_ref, out_ref):
    ht = ht_ref[...]
    p = lax.dot_general(ht, wbig_ref[...], (((1,), (0,)), ((), ())),
                        preferred_element_type=jnp.float32)
    tau = tau_ref[...]
    kp = kp_ref[...]
    x = kp + tau * (p + bcat_ref[...])

    rl = rl_ref[...]

    def iter_body(_, x):
        diff = lax.dot_general(x, gdt_ref[...], (((1,), (0,)), ((), ())),
                               preferred_element_type=jnp.float32)
        ss = lax.dot_general(diff * diff, s3t_ref[...],
                             (((1,), (0,)), ((), ())),
                             preferred_element_type=jnp.float32) + 1e-9
        dist = jnp.sqrt(ss)
        s = (dist - rl) * (_LAM_SCALE / 1.0) / (dist + 1e-9)
        sexp = lax.dot_general(s, e3_ref[...], (((1,), (0,)), ((), ())),
                               preferred_element_type=jnp.float32)
        corr = jnp.clip(sexp * diff, -_MAX_CORR, _MAX_CORR)
        return x + lax.dot_general(corr, st_ref[...],
                                   (((1,), (0,)), ((), ())),
                                   preferred_element_type=jnp.float32)

    x = lax.fori_loop(0, _ITERS, iter_body, x)
    out_ref[...] = (x - kp) / tau


_fused_call = pl.pallas_call(
    _fused_body,
    grid=(_NBLK,),
    in_specs=[
        pl.BlockSpec((_CHUNK, _NN * _D), lambda i: (i, 0)),
        pl.BlockSpec((_CHUNK, _NC3), lambda i: (i, 0)),
        pl.BlockSpec((_CHUNK, 1), lambda i: (i, 0)),
        pl.BlockSpec((_NN * _D, _NC3), lambda i: (0, 0)),
        pl.BlockSpec((1, _NC3), lambda i: (0, 0)),
        pl.BlockSpec((1, _NE), lambda i: (0, 0)),
        pl.BlockSpec((_NC3, 3 * _NE), lambda i: (0, 0)),
        pl.BlockSpec((3 * _NE, _NE), lambda i: (0, 0)),
        pl.BlockSpec((_NE, 3 * _NE), lambda i: (0, 0)),
        pl.BlockSpec((3 * _NE, _NC3), lambda i: (0, 0)),
    ],
    out_specs=pl.BlockSpec((_CHUNK, _NC3), lambda i: (i, 0)),
    out_shape=jax.ShapeDtypeStruct((_B, _NC3), jnp.float32),
    compiler_params=pltpu.CompilerParams(
        dimension_semantics=("parallel",),
        vmem_limit_bytes=100 * 1024 * 1024,
    ),
)


def kernel(model, keypoints, timesteps, hand_tokens_out, W, b, edge_index, rest_lengths):
    del model, edge_index
    W = W.astype(jnp.float32)
    tau = jnp.clip(1.0 - timesteps.astype(jnp.float32), 1e-3, None).reshape(_B, 1)
    kp = keypoints.astype(jnp.float32).reshape(_B, _NC3)
    wbig = jnp.einsum('nm,dc->ndmc', jnp.eye(_NN, dtype=jnp.float32),
                      W.T).reshape(_NN * _D, _NC3)
    bcat = jnp.tile(b.astype(jnp.float32), _NN).reshape(1, _NC3)
    rl = rest_lengths.astype(jnp.float32).reshape(1, _NE)
    ht2d = hand_tokens_out.reshape(_B, _NN * _D)
    v = _fused_call(ht2d, kp, tau, wbig, bcat, rl,
                    _GDT, _S3T, _E3, _ST)
    return v.reshape(_B, _NN, 3)


# speedup vs baseline: 1.4608x; 1.4608x over previous
import functools

import numpy as np
import jax
import jax.numpy as jnp
from jax import lax
from jax.experimental import pallas as pl
from jax.experimental.pallas import tpu as pltpu

_EDGES_I = [0, 1, 2, 3, 0, 5, 6, 7, 0, 9, 10, 11, 0, 13, 14, 15, 0, 17, 18, 19,
            21, 22, 23, 24, 21, 26, 27, 28, 21, 30, 31, 32, 21, 34, 35, 36, 21, 38, 39, 40]
_EDGES_J = [1, 2, 3, 4, 5, 6, 7, 8, 9, 10, 11, 12, 13, 14, 15, 16, 17, 18, 19, 20,
            22, 23, 24, 25, 26, 27, 28, 29, 30, 31, 32, 33, 34, 35, 36, 37, 38, 39, 40, 41]

_NE = len(_EDGES_I)
_NN = 42
_NC3 = _NN * 3
_B = 4096
_D = 512
_CHUNK = 128
_NBLK = _B // _CHUNK
_ITERS = 4
_MAX_CORR = 0.15
_LAM_SCALE = -1.0 / (2.0 + 1e-9)


def _edge_constants():
    gdt = np.zeros((_NC3, 3 * _NE), np.float32)
    s3t = np.zeros((3 * _NE, _NE), np.float32)
    e3 = np.zeros((_NE, 3 * _NE), np.float32)
    st = np.zeros((3 * _NE, _NC3), np.float32)
    for e in range(_NE):
        i, j = _EDGES_I[e], _EDGES_J[e]
        for c in range(3):
            gdt[3 * i + c, 3 * e + c] = 1.0
            gdt[3 * j + c, 3 * e + c] = -1.0
            s3t[3 * e + c, e] = 1.0
            e3[e, 3 * e + c] = 1.0
            st[3 * e + c, 3 * i + c] = -1.0
            st[3 * e + c, 3 * j + c] = 1.0
    return gdt, s3t, e3, st


_GDT, _S3T, _E3, _ST = _edge_constants()


def _fused_body(ht_ref, kp_ref, tau_ref, wbig_ref, bcat_ref, rl_ref,
                gdt_ref, s3t_ref, e3_ref, st_ref, out_ref):
    p = jnp.zeros((_CHUNK, _NC3), jnp.float32)
    for n in range(_NN):
        p = p + lax.dot_general(ht_ref[:, n, :],
                                wbig_ref[n * _D:(n + 1) * _D, :],
                                (((1,), (0,)), ((), ())),
                                preferred_element_type=jnp.float32)
    tau = tau_ref[...]
    kp = kp_ref[...]
    x = kp + tau * (p + bcat_ref[...])

    rl = rl_ref[...]

    def iter_body(_, x):
        diff = lax.dot_general(x, gdt_ref[...], (((1,), (0,)), ((), ())),
                               preferred_element_type=jnp.float32)
        ss = lax.dot_general(diff * diff, s3t_ref[...],
                             (((1,), (0,)), ((), ())),
                             preferred_element_type=jnp.float32) + 1e-9
        dist = jnp.sqrt(ss)
        s = (dist - rl) * (_LAM_SCALE / 1.0) / (dist + 1e-9)
        sexp = lax.dot_general(s, e3_ref[...], (((1,), (0,)), ((), ())),
                               preferred_element_type=jnp.float32)
        corr = jnp.clip(sexp * diff, -_MAX_CORR, _MAX_CORR)
        return x + lax.dot_general(corr, st_ref[...],
                                   (((1,), (0,)), ((), ())),
                                   preferred_element_type=jnp.float32)

    x = lax.fori_loop(0, _ITERS, iter_body, x)
    out_ref[...] = (x - kp) / tau


_fused_call = pl.pallas_call(
    _fused_body,
    grid=(_NBLK,),
    in_specs=[
        pl.BlockSpec((_CHUNK, _NN, _D), lambda i: (i, 0, 0)),
        pl.BlockSpec((_CHUNK, _NC3), lambda i: (i, 0)),
        pl.BlockSpec((_CHUNK, 1), lambda i: (i, 0)),
        pl.BlockSpec((_NN * _D, _NC3), lambda i: (0, 0)),
        pl.BlockSpec((1, _NC3), lambda i: (0, 0)),
        pl.BlockSpec((1, _NE), lambda i: (0, 0)),
        pl.BlockSpec((_NC3, 3 * _NE), lambda i: (0, 0)),
        pl.BlockSpec((3 * _NE, _NE), lambda i: (0, 0)),
        pl.BlockSpec((_NE, 3 * _NE), lambda i: (0, 0)),
        pl.BlockSpec((3 * _NE, _NC3), lambda i: (0, 0)),
    ],
    out_specs=pl.BlockSpec((_CHUNK, _NC3), lambda i: (i, 0)),
    out_shape=jax.ShapeDtypeStruct((_B, _NC3), jnp.float32),
    compiler_params=pltpu.CompilerParams(
        dimension_semantics=("parallel",),
        vmem_limit_bytes=100 * 1024 * 1024,
    ),
)


def kernel(model, keypoints, timesteps, hand_tokens_out, W, b, edge_index, rest_lengths):
    del model, edge_index
    W = W.astype(jnp.float32)
    tau = jnp.clip(1.0 - timesteps.astype(jnp.float32), 1e-3, None).reshape(_B, 1)
    kp = keypoints.astype(jnp.float32).reshape(_B, _NC3)
    wbig = jnp.einsum('nm,dc->ndmc', jnp.eye(_NN, dtype=jnp.float32),
                      W.T).reshape(_NN * _D, _NC3)
    bcat = jnp.tile(b.astype(jnp.float32), _NN).reshape(1, _NC3)
    rl = rest_lengths.astype(jnp.float32).reshape(1, _NE)
    v = _fused_call(hand_tokens_out, kp, tau, wbig, bcat, rl,
                    _GDT, _S3T, _E3, _ST)
    return v.reshape(_B, _NN, 3)


# speedup vs baseline: 1.5994x; 1.0948x over previous
import functools

import numpy as np
import jax
import jax.numpy as jnp
from jax import lax
from jax.experimental import pallas as pl
from jax.experimental.pallas import tpu as pltpu

_EDGES_I = [0, 1, 2, 3, 0, 5, 6, 7, 0, 9, 10, 11, 0, 13, 14, 15, 0, 17, 18, 19,
            21, 22, 23, 24, 21, 26, 27, 28, 21, 30, 31, 32, 21, 34, 35, 36, 21, 38, 39, 40]
_EDGES_J = [1, 2, 3, 4, 5, 6, 7, 8, 9, 10, 11, 12, 13, 14, 15, 16, 17, 18, 19, 20,
            22, 23, 24, 25, 26, 27, 28, 29, 30, 31, 32, 33, 34, 35, 36, 37, 38, 39, 40, 41]

_NE = len(_EDGES_I)
_NN = 42
_NC3 = _NN * 3
_B = 4096
_D = 512
_CHUNK = 256
_NBLK = _B // _CHUNK
_ITERS = 4
_MAX_CORR = 0.15
_LAM_SCALE = -1.0 / (2.0 + 1e-9)


def _edge_constants():
    gdt = np.zeros((_NC3, 3 * _NE), np.float32)
    s3t = np.zeros((3 * _NE, _NE), np.float32)
    e3 = np.zeros((_NE, 3 * _NE), np.float32)
    st = np.zeros((3 * _NE, _NC3), np.float32)
    for e in range(_NE):
        i, j = _EDGES_I[e], _EDGES_J[e]
        for c in range(3):
            gdt[3 * i + c, 3 * e + c] = 1.0
            gdt[3 * j + c, 3 * e + c] = -1.0
            s3t[3 * e + c, e] = 1.0
            e3[e, 3 * e + c] = 1.0
            st[3 * e + c, 3 * i + c] = -1.0
            st[3 * e + c, 3 * j + c] = 1.0
    return gdt, s3t, e3, st


_GDT, _S3T, _E3, _ST = _edge_constants()


def _fused_body(ht_ref, kp_ref, tau_ref, wbig_ref, bcat_ref, rl_ref,
                gdt_ref, s3t_ref, e3_ref, st_ref, out_ref):
    p = jnp.zeros((_CHUNK, _NC3), jnp.float32)
    for n in range(_NN):
        p = p + lax.dot_general(ht_ref[:, n, :],
                                wbig_ref[n * _D:(n + 1) * _D, :],
                                (((1,), (0,)), ((), ())),
                                preferred_element_type=jnp.float32)
    tau = tau_ref[...]
    kp = kp_ref[...]
    x = kp + tau * (p + bcat_ref[...])

    rl = rl_ref[...]

    def iter_body(_, x):
        diff = lax.dot_general(x, gdt_ref[...], (((1,), (0,)), ((), ())),
                               preferred_element_type=jnp.float32)
        ss = lax.dot_general(diff * diff, s3t_ref[...],
                             (((1,), (0,)), ((), ())),
                             preferred_element_type=jnp.float32) + 1e-9
        dist = jnp.sqrt(ss)
        s = (dist - rl) * (_LAM_SCALE / 1.0) / (dist + 1e-9)
        sexp = lax.dot_general(s, e3_ref[...], (((1,), (0,)), ((), ())),
                               preferred_element_type=jnp.float32)
        corr = jnp.clip(sexp * diff, -_MAX_CORR, _MAX_CORR)
        return x + lax.dot_general(corr, st_ref[...],
                                   (((1,), (0,)), ((), ())),
                                   preferred_element_type=jnp.float32)

    x = lax.fori_loop(0, _ITERS, iter_body, x)
    out_ref[...] = (x - kp) / tau


_fused_call = pl.pallas_call(
    _fused_body,
    grid=(_NBLK,),
    in_specs=[
        pl.BlockSpec((_CHUNK, _NN, _D), lambda i: (i, 0, 0)),
        pl.BlockSpec((_CHUNK, _NC3), lambda i: (i, 0)),
        pl.BlockSpec((_CHUNK, 1), lambda i: (i, 0)),
        pl.BlockSpec((_NN * _D, _NC3), lambda i: (0, 0)),
        pl.BlockSpec((1, _NC3), lambda i: (0, 0)),
        pl.BlockSpec((1, _NE), lambda i: (0, 0)),
        pl.BlockSpec((_NC3, 3 * _NE), lambda i: (0, 0)),
        pl.BlockSpec((3 * _NE, _NE), lambda i: (0, 0)),
        pl.BlockSpec((_NE, 3 * _NE), lambda i: (0, 0)),
        pl.BlockSpec((3 * _NE, _NC3), lambda i: (0, 0)),
    ],
    out_specs=pl.BlockSpec((_CHUNK, _NC3), lambda i: (i, 0)),
    out_shape=jax.ShapeDtypeStruct((_B, _NC3), jnp.float32),
    compiler_params=pltpu.CompilerParams(
        dimension_semantics=("parallel",),
        vmem_limit_bytes=100 * 1024 * 1024,
    ),
)


def kernel(model, keypoints, timesteps, hand_tokens_out, W, b, edge_index, rest_lengths):
    del model, edge_index
    W = W.astype(jnp.float32)
    tau = jnp.clip(1.0 - timesteps.astype(jnp.float32), 1e-3, None).reshape(_B, 1)
    kp = keypoints.astype(jnp.float32).reshape(_B, _NC3)
    wbig = jnp.einsum('nm,dc->ndmc', jnp.eye(_NN, dtype=jnp.float32),
                      W.T).reshape(_NN * _D, _NC3)
    bcat = jnp.tile(b.astype(jnp.float32), _NN).reshape(1, _NC3)
    rl = rest_lengths.astype(jnp.float32).reshape(1, _NE)
    v = _fused_call(hand_tokens_out, kp, tau, wbig, bcat, rl,
                    _GDT, _S3T, _E3, _ST)
    return v.reshape(_B, _NN, 3)


# speedup vs baseline: 1.6024x; 1.0019x over previous
import functools

import numpy as np
import jax
import jax.numpy as jnp
from jax import lax
from jax.experimental import pallas as pl
from jax.experimental.pallas import tpu as pltpu

_EDGES_I = [0, 1, 2, 3, 0, 5, 6, 7, 0, 9, 10, 11, 0, 13, 14, 15, 0, 17, 18, 19,
            21, 22, 23, 24, 21, 26, 27, 28, 21, 30, 31, 32, 21, 34, 35, 36, 21, 38, 39, 40]
_EDGES_J = [1, 2, 3, 4, 5, 6, 7, 8, 9, 10, 11, 12, 13, 14, 15, 16, 17, 18, 19, 20,
            22, 23, 24, 25, 26, 27, 28, 29, 30, 31, 32, 33, 34, 35, 36, 37, 38, 39, 40, 41]

_NE = len(_EDGES_I)
_NN = 42
_NC3 = _NN * 3
_B = 4096
_D = 512
_CHUNK = 256
_NBLK = _B // _CHUNK
_ITERS = 4
_MAX_CORR = 0.15
_LAM_SCALE = -1.0 / (2.0 + 1e-9)


def _edge_constants():
    gdt = np.zeros((_NC3, 3 * _NE), np.float32)
    s3t = np.zeros((3 * _NE, _NE), np.float32)
    e3 = np.zeros((_NE, 3 * _NE), np.float32)
    st = np.zeros((3 * _NE, _NC3), np.float32)
    for e in range(_NE):
        i, j = _EDGES_I[e], _EDGES_J[e]
        for c in range(3):
            gdt[3 * i + c, 3 * e + c] = 1.0
            gdt[3 * j + c, 3 * e + c] = -1.0
            s3t[3 * e + c, e] = 1.0
            e3[e, 3 * e + c] = 1.0
            st[3 * e + c, 3 * i + c] = -1.0
            st[3 * e + c, 3 * j + c] = 1.0
    return gdt, s3t, e3, st


_GDT, _S3T, _E3, _ST = _edge_constants()


def _fused_body(ht_ref, kp_ref, tau_ref, wbig_ref, bcat_ref, rl_ref,
                gdt_ref, s3t_ref, e3_ref, st_ref, out_ref):
    accs = [jnp.zeros((_CHUNK, _NC3), jnp.float32) for _ in range(4)]
    for n in range(_NN):
        accs[n % 4] = accs[n % 4] + lax.dot_general(
            ht_ref[:, n, :],
            wbig_ref[n * _D:(n + 1) * _D, :],
            (((1,), (0,)), ((), ())),
            preferred_element_type=jnp.float32)
    p = (accs[0] + accs[1]) + (accs[2] + accs[3])
    tau = tau_ref[...]
    kp = kp_ref[...]
    x = kp + tau * (p + bcat_ref[...])

    rl = rl_ref[...]

    def iter_body(_, x):
        diff = lax.dot_general(x, gdt_ref[...], (((1,), (0,)), ((), ())),
                               preferred_element_type=jnp.float32)
        ss = lax.dot_general(diff * diff, s3t_ref[...],
                             (((1,), (0,)), ((), ())),
                             preferred_element_type=jnp.float32) + 1e-9
        dist = jnp.sqrt(ss)
        s = (dist - rl) * (_LAM_SCALE / 1.0) / (dist + 1e-9)
        sexp = lax.dot_general(s, e3_ref[...], (((1,), (0,)), ((), ())),
                               preferred_element_type=jnp.float32)
        corr = jnp.clip(sexp * diff, -_MAX_CORR, _MAX_CORR)
        return x + lax.dot_general(corr, st_ref[...],
                                   (((1,), (0,)), ((), ())),
                                   preferred_element_type=jnp.float32)

    x = lax.fori_loop(0, _ITERS, iter_body, x)
    out_ref[...] = (x - kp) / tau


_fused_call = pl.pallas_call(
    _fused_body,
    grid=(_NBLK,),
    in_specs=[
        pl.BlockSpec((_CHUNK, _NN, _D), lambda i: (i, 0, 0)),
        pl.BlockSpec((_CHUNK, _NC3), lambda i: (i, 0)),
        pl.BlockSpec((_CHUNK, 1), lambda i: (i, 0)),
        pl.BlockSpec((_NN * _D, _NC3), lambda i: (0, 0)),
        pl.BlockSpec((1, _NC3), lambda i: (0, 0)),
        pl.BlockSpec((1, _NE), lambda i: (0, 0)),
        pl.BlockSpec((_NC3, 3 * _NE), lambda i: (0, 0)),
        pl.BlockSpec((3 * _NE, _NE), lambda i: (0, 0)),
        pl.BlockSpec((_NE, 3 * _NE), lambda i: (0, 0)),
        pl.BlockSpec((3 * _NE, _NC3), lambda i: (0, 0)),
    ],
    out_specs=pl.BlockSpec((_CHUNK, _NC3), lambda i: (i, 0)),
    out_shape=jax.ShapeDtypeStruct((_B, _NC3), jnp.float32),
    compiler_params=pltpu.CompilerParams(
        dimension_semantics=("parallel",),
        vmem_limit_bytes=64 * 1024 * 1024,
    ),
)


def kernel(model, keypoints, timesteps, hand_tokens_out, W, b, edge_index, rest_lengths):
    del model, edge_index
    W = W.astype(jnp.float32)
    tau = jnp.clip(1.0 - timesteps.astype(jnp.float32), 1e-3, None).reshape(_B, 1)
    kp = keypoints.astype(jnp.float32).reshape(_B, _NC3)
    wbig = jnp.einsum('nm,dc->ndmc', jnp.eye(_NN, dtype=jnp.float32),
                      W.T).reshape(_NN * _D, _NC3)
    bcat = jnp.tile(b.astype(jnp.float32), _NN).reshape(1, _NC3)
    rl = rest_lengths.astype(jnp.float32).reshape(1, _NE)
    v = _fused_call(hand_tokens_out, kp, tau, wbig, bcat, rl,
                    _GDT, _S3T, _E3, _ST)
    return v.reshape(_B, _NN, 3)
